# baseline (speedup 1.0000x reference)
import functools

import jax
import jax.numpy as jnp
import numpy as np
from jax import lax
from jax.experimental import pallas as pl
from jax.experimental.pallas import tpu as pltpu
from jax.experimental.pallas import tpu_sc as plsc

B, N, K, C, O = 8, 4096, 32, 64, 64
LANES = 16
NW = 32
GB = 4
NG = B // GB
PPW = GB * N // NW
WPB = N // PPW

NBUF = 4
RPD = 128
PPD = RPD // K
NDESC = PPW // PPD
NTURN = NDESC // NBUF


def _mm_body(x_ref, pos_ref, wfp_ref, wpp_ref, wp_ref, b_ref, z_ref, c_ref):
    posb = pos_ref[0]
    zp = (jnp.dot(x_ref[0], wfp_ref[...],
                  preferred_element_type=jnp.float32)
          + jnp.dot(posb, wpp_ref[...],
                    preferred_element_type=jnp.float32)
          + b_ref[...])
    z_ref[0] = zp.astype(jnp.bfloat16)
    c_ref[0] = jnp.dot(posb, wp_ref[...],
                       preferred_element_type=jnp.float32)


def _make_tc_mm(g):
    return pl.pallas_call(
        _mm_body,
        grid=(GB,),
        in_specs=[
            pl.BlockSpec((1, N, C), lambda i: (g * GB + i, 0, 0)),
            pl.BlockSpec((1, N, 3), lambda i: (g * GB + i, 0, 0)),
            pl.BlockSpec((C, O), lambda i: (0, 0)),
            pl.BlockSpec((3, O), lambda i: (0, 0)),
            pl.BlockSpec((3, O), lambda i: (0, 0)),
            pl.BlockSpec((1, O), lambda i: (0, 0)),
        ],
        out_specs=[
            pl.BlockSpec((1, N, O), lambda i: (i, 0, 0)),
            pl.BlockSpec((1, N, O), lambda i: (i, 0, 0)),
        ],
        out_shape=[
            jax.ShapeDtypeStruct((GB, N, O), jnp.bfloat16),
            jax.ShapeDtypeStruct((GB, N, O), jnp.float32),
        ],
    )


_TC_MM = [_make_tc_mm(g) for g in range(NG)]


def _sc_body(z_hbm, c_hbm, idx_hbm, out_hbm, idx_all, rows_v, c_v, out_v,
             semg0, semg1, semg2, semg3, semo0, semo1, semo2, semo3):
    nc = 2
    wid = lax.axis_index("s") * nc + lax.axis_index("c")
    bb = wid // WPB
    lp_base = (wid % WPB) * PPW
    ztab = z_hbm.at[bb]
    ctab = c_hbm.at[bb]
    otab = out_hbm.at[bb]
    semg = (semg0, semg1, semg2, semg3)
    semo = (semo0, semo1, semo2, semo3)

    pltpu.sync_copy(idx_hbm.at[wid], idx_all)

    def fire(d, b):
        pltpu.async_copy(ztab.at[idx_all.at[d]], rows_v.at[b], semg[b])
        pltpu.async_copy(ctab.at[pl.ds(lp_base + d * PPD, PPD)], c_v.at[b],
                         semg[b])

    def drain(b):
        pltpu.make_async_copy(ztab.at[idx_all.at[0]], rows_v.at[b],
                              semg[b]).wait()
        pltpu.make_async_copy(ctab.at[pl.ds(0, PPD)], c_v.at[b],
                              semg[b]).wait()

    def fire_out(d, b):
        pltpu.async_copy(out_v.at[b], otab.at[pl.ds(lp_base + d * PPD, PPD)],
                         semo[b])

    def drain_out(b):
        pltpu.make_async_copy(out_v.at[b], otab.at[pl.ds(0, PPD)],
                              semo[b]).wait()

    def compute(b):
        for t in range(PPD):
            rr = t * K
            for g in range(O // 32):
                sl = pl.ds(g * 32, 32)
                acc = rows_v[b, rr, sl]
                for k in range(1, K):
                    acc = jnp.maximum(acc, rows_v[b, rr + k, sl])
                lo, hi = plsc.unpack(acc, format=plsc.PackFormat.INTERLEAVED)
                sl_lo = pl.ds(g * 32, LANES)
                sl_hi = pl.ds(g * 32 + LANES, LANES)
                out_v[b, t, sl_lo] = jnp.maximum(lo - c_v[b, t, sl_lo], 0.0)
                out_v[b, t, sl_hi] = jnp.maximum(hi - c_v[b, t, sl_hi], 0.0)

    for b in range(NBUF):
        fire(b, b)

    def turn(q, carry):
        for b in range(NBUF):
            d = NBUF * q + b
            drain(b)

            @pl.when(d >= NBUF)
            def _():
                drain_out(b)

            compute(b)
            fire_out(d, b)

            @pl.when(d + NBUF < NDESC)
            def _():
                fire(d + NBUF, b)
        return carry

    lax.fori_loop(0, NTURN, turn, 0)
    for b in range(NBUF):
        drain_out(b)


_sc_gathermax = functools.partial(
    pl.kernel,
    out_type=jax.ShapeDtypeStruct((GB, N, O), jnp.float32),
    mesh=plsc.VectorSubcoreMesh(core_axis_name="c", subcore_axis_name="s"),
    scratch_types=[
        pltpu.VMEM((NDESC, RPD), jnp.int32),
        pltpu.VMEM((NBUF, RPD, O), jnp.bfloat16),
        pltpu.VMEM((NBUF, PPD, O), jnp.float32),
        pltpu.VMEM((NBUF, PPD, O), jnp.float32),
        pltpu.SemaphoreType.DMA,
        pltpu.SemaphoreType.DMA,
        pltpu.SemaphoreType.DMA,
        pltpu.SemaphoreType.DMA,
        pltpu.SemaphoreType.DMA,
        pltpu.SemaphoreType.DMA,
        pltpu.SemaphoreType.DMA,
        pltpu.SemaphoreType.DMA,
    ],
    compiler_params=pltpu.CompilerParams(use_tc_tiling_on_sc=False,
                                         needs_layout_passes=False),
)(_sc_body)


_PERM = np.empty(O, dtype=np.int32)
for _g in range(O // 32):
    _PERM[_g * 32 + 0:_g * 32 + 32:2] = np.arange(16) + _g * 32
    _PERM[_g * 32 + 1:_g * 32 + 32:2] = np.arange(16) + _g * 32 + 16


def kernel(x, pos, neighbor_idx, W, b):
    wf = W[3:]
    wp = W[:3]
    wfp = wf[:, _PERM]
    wpp = wp[:, _PERM]
    bp = b[_PERM].reshape(1, O)
    outs = []
    for g in range(NG):
        idx_g = lax.slice_in_dim(neighbor_idx, g * GB, (g + 1) * GB
                                 ).reshape(NW, NDESC, RPD)
        z, c = _TC_MM[g](x, pos, wfp, wpp, wp, bp)
        outs.append(_sc_gathermax(z, c, idx_g))
    return jnp.concatenate(outs, axis=0)


# speedup vs baseline: 1.0139x; 1.0139x over previous
import functools

import jax
import jax.numpy as jnp
import numpy as np
from jax import lax
from jax.experimental import pallas as pl
from jax.experimental.pallas import tpu as pltpu
from jax.experimental.pallas import tpu_sc as plsc

B, N, K, C, O = 8, 4096, 32, 64, 64
LANES = 16
NW = 32
GB = 2
NG = B // GB
PPW = GB * N // NW
WPB = N // PPW

NBUF = 4
RPD = 128
PPD = RPD // K
NDESC = PPW // PPD
NTURN = NDESC // NBUF


def _mm_body(x_ref, pos_ref, wfp_ref, wpp_ref, wp_ref, b_ref, z_ref, c_ref):
    posb = pos_ref[0]
    zp = (jnp.dot(x_ref[0], wfp_ref[...],
                  preferred_element_type=jnp.float32)
          + jnp.dot(posb, wpp_ref[...],
                    preferred_element_type=jnp.float32)
          + b_ref[...])
    z_ref[0] = zp.astype(jnp.bfloat16)
    c_ref[0] = jnp.dot(posb, wp_ref[...],
                       preferred_element_type=jnp.float32)


def _make_tc_mm(g):
    return pl.pallas_call(
        _mm_body,
        grid=(GB,),
        in_specs=[
            pl.BlockSpec((1, N, C), lambda i: (g * GB + i, 0, 0)),
            pl.BlockSpec((1, N, 3), lambda i: (g * GB + i, 0, 0)),
            pl.BlockSpec((C, O), lambda i: (0, 0)),
            pl.BlockSpec((3, O), lambda i: (0, 0)),
            pl.BlockSpec((3, O), lambda i: (0, 0)),
            pl.BlockSpec((1, O), lambda i: (0, 0)),
        ],
        out_specs=[
            pl.BlockSpec((1, N, O), lambda i: (i, 0, 0)),
            pl.BlockSpec((1, N, O), lambda i: (i, 0, 0)),
        ],
        out_shape=[
            jax.ShapeDtypeStruct((GB, N, O), jnp.bfloat16),
            jax.ShapeDtypeStruct((GB, N, O), jnp.float32),
        ],
    )


_TC_MM = [_make_tc_mm(g) for g in range(NG)]


def _sc_body(z_hbm, c_hbm, idx_hbm, out_hbm, idx_all, rows_v, c_v, out_v,
             semg0, semg1, semg2, semg3, semo0, semo1, semo2, semo3):
    nc = 2
    wid = lax.axis_index("s") * nc + lax.axis_index("c")
    bb = wid // WPB
    lp_base = (wid % WPB) * PPW
    ztab = z_hbm.at[bb]
    ctab = c_hbm.at[bb]
    otab = out_hbm.at[bb]
    semg = (semg0, semg1, semg2, semg3)
    semo = (semo0, semo1, semo2, semo3)

    pltpu.sync_copy(idx_hbm.at[wid], idx_all)

    def fire(d, b):
        pltpu.async_copy(ztab.at[idx_all.at[d]], rows_v.at[b], semg[b])
        pltpu.async_copy(ctab.at[pl.ds(lp_base + d * PPD, PPD)], c_v.at[b],
                         semg[b])

    def drain(b):
        pltpu.make_async_copy(ztab.at[idx_all.at[0]], rows_v.at[b],
                              semg[b]).wait()
        pltpu.make_async_copy(ctab.at[pl.ds(0, PPD)], c_v.at[b],
                              semg[b]).wait()

    def fire_out(d, b):
        pltpu.async_copy(out_v.at[b], otab.at[pl.ds(lp_base + d * PPD, PPD)],
                         semo[b])

    def drain_out(b):
        pltpu.make_async_copy(out_v.at[b], otab.at[pl.ds(0, PPD)],
                              semo[b]).wait()

    def compute(b):
        for t in range(PPD):
            rr = t * K
            for g in range(O // 32):
                sl = pl.ds(g * 32, 32)
                acc = rows_v[b, rr, sl]
                for k in range(1, K):
                    acc = jnp.maximum(acc, rows_v[b, rr + k, sl])
                lo, hi = plsc.unpack(acc, format=plsc.PackFormat.INTERLEAVED)
                sl_lo = pl.ds(g * 32, LANES)
                sl_hi = pl.ds(g * 32 + LANES, LANES)
                out_v[b, t, sl_lo] = jnp.maximum(lo - c_v[b, t, sl_lo], 0.0)
                out_v[b, t, sl_hi] = jnp.maximum(hi - c_v[b, t, sl_hi], 0.0)

    for b in range(NBUF):
        fire(b, b)

    def turn(q, carry):
        for b in range(NBUF):
            d = NBUF * q + b
            drain(b)

            @pl.when(d >= NBUF)
            def _():
                drain_out(b)

            compute(b)
            fire_out(d, b)

            @pl.when(d + NBUF < NDESC)
            def _():
                fire(d + NBUF, b)
        return carry

    lax.fori_loop(0, NTURN, turn, 0)
    for b in range(NBUF):
        drain_out(b)


_sc_gathermax = functools.partial(
    pl.kernel,
    out_type=jax.ShapeDtypeStruct((GB, N, O), jnp.float32),
    mesh=plsc.VectorSubcoreMesh(core_axis_name="c", subcore_axis_name="s"),
    scratch_types=[
        pltpu.VMEM((NDESC, RPD), jnp.int32),
        pltpu.VMEM((NBUF, RPD, O), jnp.bfloat16),
        pltpu.VMEM((NBUF, PPD, O), jnp.float32),
        pltpu.VMEM((NBUF, PPD, O), jnp.float32),
        pltpu.SemaphoreType.DMA,
        pltpu.SemaphoreType.DMA,
        pltpu.SemaphoreType.DMA,
        pltpu.SemaphoreType.DMA,
        pltpu.SemaphoreType.DMA,
        pltpu.SemaphoreType.DMA,
        pltpu.SemaphoreType.DMA,
        pltpu.SemaphoreType.DMA,
    ],
    compiler_params=pltpu.CompilerParams(use_tc_tiling_on_sc=False,
                                         needs_layout_passes=False),
)(_sc_body)


_PERM = np.empty(O, dtype=np.int32)
for _g in range(O // 32):
    _PERM[_g * 32 + 0:_g * 32 + 32:2] = np.arange(16) + _g * 32
    _PERM[_g * 32 + 1:_g * 32 + 32:2] = np.arange(16) + _g * 32 + 16


def kernel(x, pos, neighbor_idx, W, b):
    wf = W[3:]
    wp = W[:3]
    wfp = wf[:, _PERM]
    wpp = wp[:, _PERM]
    bp = b[_PERM].reshape(1, O)
    outs = []
    for g in range(NG):
        idx_g = lax.slice_in_dim(neighbor_idx, g * GB, (g + 1) * GB
                                 ).reshape(NW, NDESC, RPD)
        z, c = _TC_MM[g](x, pos, wfp, wpp, wp, bp)
        outs.append(_sc_gathermax(z, c, idx_g))
    return jnp.concatenate(outs, axis=0)
